# baseline (speedup 1.0000x reference)
import functools

import jax
import jax.numpy as jnp
from jax.experimental import pallas as pl
from jax.experimental.pallas import tpu as pltpu
from jax.experimental.pallas import tpu_sc as plsc

_N_SPARSE = 26
_VOCAB = 100000
_EMB = 16
_B = 16384
_DENSE = 13
_H1, _H2, _OUT = 256, 128, 64
_FPAD = 32
_NQ = 4
_BGRP = 16
_NWIN = _NQ * (_B // _BGRP)
_BM = 1024


def _sc_gather4(tab_flat, idx_win):
    mesh = plsc.VectorSubcoreMesh(core_axis_name="core", subcore_axis_name="subcore")

    @functools.partial(
        pl.kernel,
        out_type=jax.ShapeDtypeStruct((_NQ * _B, 8 * _EMB), jnp.float32),
        mesh=mesh,
        scratch_types=[pltpu.VMEM((128, _EMB), jnp.float32),
                       pltpu.SemaphoreType.DMA],
        compiler_params=pltpu.CompilerParams(use_tc_tiling_on_sc=False),
    )
    def k(tab_hbm, idx_hbm, o_hbm, gbuf, sem):
        def body(i_vmem, o_vmem):
            pltpu.sync_copy(tab_hbm.at[i_vmem.at[0]], gbuf)
            for j in range(_BGRP):
                for s in range(8):
                    o_vmem[j, pl.ds(s * _EMB, _EMB)] = gbuf[s * _BGRP + j, :]

        pltpu.emit_pipeline(
            body,
            grid=(_NWIN,),
            in_specs=[pl.BlockSpec((1, 128), index_map=lambda w: (0, w))],
            out_specs=[pl.BlockSpec((_BGRP, 8 * _EMB),
                                    index_map=lambda w: (w, 0))],
            core_axis_name=("core", "subcore"),
            dimension_semantics=(pltpu.PARALLEL,),
        )(idx_hbm, o_hbm)

    return k(tab_flat, idx_win)


def _mlp_body(x4_ref, dense_ref, w1e_ref, w1b_ref, b1_ref, w2_ref, b2_ref,
              wo_ref, bo_ref, o_ref):
    dn = (((1,), (0,)), ((), ()))
    f32 = jnp.float32
    h = jax.lax.dot_general(x4_ref[0], w1e_ref[pl.ds(0, 128), :], dn,
                            preferred_element_type=f32)
    for q in range(1, _NQ):
        h = h + jax.lax.dot_general(x4_ref[q], w1e_ref[pl.ds(128 * q, 128), :],
                                    dn, preferred_element_type=f32)
    h = h + jax.lax.dot_general(dense_ref[...], w1b_ref[...], dn,
                                preferred_element_type=f32)
    h = jnp.maximum(h + b1_ref[...], 0.0)
    h = jax.lax.dot_general(h, w2_ref[...], dn, preferred_element_type=f32)
    h = jnp.maximum(h + b2_ref[...], 0.0)
    out = jax.lax.dot_general(h, wo_ref[...], dn, preferred_element_type=f32)
    out = out + bo_ref[...]
    ssq = jnp.sum(out * out, axis=1, keepdims=True)
    denom = jnp.maximum(jnp.sqrt(ssq), 1e-12)
    o_ref[...] = out / denom


def _tc_mlp(x4, dense_0, W1, b1, W2, b2, Wout, bout):
    w1a = W1[:_N_SPARSE * _EMB].reshape(_N_SPARSE, _EMB, _H1)
    w1e = jnp.pad(w1a, ((0, _FPAD - _N_SPARSE), (0, 0), (0, 0))).reshape(
        _FPAD * _EMB, _H1)
    w1b = W1[_N_SPARSE * _EMB:]
    full = lambda shape: pl.BlockSpec(shape, lambda i: tuple(0 for _ in shape))
    return pl.pallas_call(
        _mlp_body,
        grid=(_B // _BM,),
        in_specs=[
            pl.BlockSpec((_NQ, _BM, 128), lambda i: (0, i, 0)),
            pl.BlockSpec((_BM, _DENSE), lambda i: (i, 0)),
            full((_FPAD * _EMB, _H1)),
            full((_DENSE, _H1)),
            full((1, _H1)),
            full((_H1, _H2)),
            full((1, _H2)),
            full((_H2, _OUT)),
            full((1, _OUT)),
        ],
        out_specs=pl.BlockSpec((_BM, _OUT), lambda i: (i, 0)),
        out_shape=jax.ShapeDtypeStruct((_B, _OUT), jnp.float32),
    )(x4, dense_0, w1e, w1b, b1[None, :], W2, b2[None, :], Wout, bout[None, :])


def kernel(sparse_0, sparse_1, sparse_2, sparse_3, sparse_4, sparse_5,
           sparse_6, sparse_7, sparse_8, sparse_9, sparse_10, sparse_11,
           sparse_12, sparse_13, sparse_14, sparse_15, sparse_16, sparse_17,
           sparse_18, sparse_19, sparse_20, sparse_21, sparse_22, sparse_23,
           sparse_24, sparse_25, dense_0, tables, W1, b1, W2, b2, Wout, bout):
    sparse = [sparse_0, sparse_1, sparse_2, sparse_3, sparse_4, sparse_5,
              sparse_6, sparse_7, sparse_8, sparse_9, sparse_10, sparse_11,
              sparse_12, sparse_13, sparse_14, sparse_15, sparse_16,
              sparse_17, sparse_18, sparse_19, sparse_20, sparse_21,
              sparse_22, sparse_23, sparse_24, sparse_25]
    idx = jnp.stack(sparse, axis=1)
    offs = (jnp.arange(_N_SPARSE, dtype=jnp.int32) * _VOCAB)[None, :]
    idx = jnp.pad(idx + offs, ((0, 0), (0, _FPAD - _N_SPARSE)))
    idx_win = idx.reshape(_B // _BGRP, _BGRP, _NQ, 8).transpose(2, 0, 3, 1)
    idx_win = idx_win.reshape(1, _NWIN * 128)
    tab_flat = tables.reshape(_N_SPARSE * _VOCAB, _EMB)
    x4 = _sc_gather4(tab_flat, idx_win).reshape(_NQ, _B, 8 * _EMB)
    return _tc_mlp(x4, dense_0, W1, b1, W2, b2, Wout, bout)


# speedup vs baseline: 1.3460x; 1.3460x over previous
import functools

import jax
import jax.numpy as jnp
from jax.experimental import pallas as pl
from jax.experimental.pallas import tpu as pltpu
from jax.experimental.pallas import tpu_sc as plsc

_N_SPARSE = 26
_VOCAB = 100000
_EMB = 16
_B = 16384
_DENSE = 13
_H1, _H2, _OUT = 256, 128, 64
_FPAD = 32
_NQ = 4
_BGRP = 16
_NWIN = _NQ * (_B // _BGRP)
_BM = 1024


def _sc_gather4(tab_flat, idx_win):
    mesh = plsc.VectorSubcoreMesh(core_axis_name="core", subcore_axis_name="subcore")

    @functools.partial(
        pl.kernel,
        out_type=jax.ShapeDtypeStruct((_NQ, _B, 8 * _EMB), jnp.float32),
        mesh=mesh,
        scratch_types=[pltpu.VMEM((_N_SPARSE, _BGRP, _EMB), jnp.float32),
                       pltpu.SemaphoreType.DMA],
        compiler_params=pltpu.CompilerParams(use_tc_tiling_on_sc=False),
    )
    def k(tab_hbm, idx_hbm, o_hbm, gbuf, sem):
        zeros = jnp.zeros((_EMB,), jnp.float32)

        def body(i_vmem, o_vmem):
            handles = [
                pltpu.async_copy(
                    tab_hbm.at[f].at[i_vmem.at[0, f]], gbuf.at[f], sem)
                for f in range(_N_SPARSE)
            ]
            for h in handles:
                h.wait()
            for j in range(_BGRP):
                for f in range(_N_SPARSE):
                    o_vmem[f // 8, j, pl.ds((f % 8) * _EMB, _EMB)] = gbuf[f, j, :]
                for s in range(_N_SPARSE % 8, 8):
                    o_vmem[_NQ - 1, j, pl.ds(s * _EMB, _EMB)] = zeros

        pltpu.emit_pipeline(
            body,
            grid=(_B // _BGRP,),
            in_specs=[pl.BlockSpec((1, _N_SPARSE, _BGRP),
                                   index_map=lambda w: (w, 0, 0))],
            out_specs=[pl.BlockSpec((_NQ, _BGRP, 8 * _EMB),
                                    index_map=lambda w: (0, w, 0))],
            core_axis_name=("core", "subcore"),
            dimension_semantics=(pltpu.PARALLEL,),
        )(idx_hbm, o_hbm)

    return k(tab_flat, idx_win)


def _mlp_body(x4_ref, dense_ref, w1e_ref, w1b_ref, b1_ref, w2_ref, b2_ref,
              wo_ref, bo_ref, o_ref):
    dn = (((1,), (0,)), ((), ()))
    f32 = jnp.float32
    h = jax.lax.dot_general(x4_ref[0], w1e_ref[pl.ds(0, 128), :], dn,
                            preferred_element_type=f32)
    for q in range(1, _NQ):
        h = h + jax.lax.dot_general(x4_ref[q], w1e_ref[pl.ds(128 * q, 128), :],
                                    dn, preferred_element_type=f32)
    h = h + jax.lax.dot_general(dense_ref[...], w1b_ref[...], dn,
                                preferred_element_type=f32)
    h = jnp.maximum(h + b1_ref[...], 0.0)
    h = jax.lax.dot_general(h, w2_ref[...], dn, preferred_element_type=f32)
    h = jnp.maximum(h + b2_ref[...], 0.0)
    out = jax.lax.dot_general(h, wo_ref[...], dn, preferred_element_type=f32)
    out = out + bo_ref[...]
    ssq = jnp.sum(out * out, axis=1, keepdims=True)
    denom = jnp.maximum(jnp.sqrt(ssq), 1e-12)
    o_ref[...] = out / denom


def _tc_mlp(x4, dense_0, W1, b1, W2, b2, Wout, bout):
    w1a = W1[:_N_SPARSE * _EMB].reshape(_N_SPARSE, _EMB, _H1)
    w1e = jnp.pad(w1a, ((0, _FPAD - _N_SPARSE), (0, 0), (0, 0))).reshape(
        _FPAD * _EMB, _H1)
    w1b = W1[_N_SPARSE * _EMB:]
    full = lambda shape: pl.BlockSpec(shape, lambda i: tuple(0 for _ in shape))
    return pl.pallas_call(
        _mlp_body,
        grid=(_B // _BM,),
        in_specs=[
            pl.BlockSpec((_NQ, _BM, 128), lambda i: (0, i, 0)),
            pl.BlockSpec((_BM, _DENSE), lambda i: (i, 0)),
            full((_FPAD * _EMB, _H1)),
            full((_DENSE, _H1)),
            full((1, _H1)),
            full((_H1, _H2)),
            full((1, _H2)),
            full((_H2, _OUT)),
            full((1, _OUT)),
        ],
        out_specs=pl.BlockSpec((_BM, _OUT), lambda i: (i, 0)),
        out_shape=jax.ShapeDtypeStruct((_B, _OUT), jnp.float32),
    )(x4, dense_0, w1e, w1b, b1[None, :], W2, b2[None, :], Wout, bout[None, :])


def kernel(sparse_0, sparse_1, sparse_2, sparse_3, sparse_4, sparse_5,
           sparse_6, sparse_7, sparse_8, sparse_9, sparse_10, sparse_11,
           sparse_12, sparse_13, sparse_14, sparse_15, sparse_16, sparse_17,
           sparse_18, sparse_19, sparse_20, sparse_21, sparse_22, sparse_23,
           sparse_24, sparse_25, dense_0, tables, W1, b1, W2, b2, Wout, bout):
    sparse = [sparse_0, sparse_1, sparse_2, sparse_3, sparse_4, sparse_5,
              sparse_6, sparse_7, sparse_8, sparse_9, sparse_10, sparse_11,
              sparse_12, sparse_13, sparse_14, sparse_15, sparse_16,
              sparse_17, sparse_18, sparse_19, sparse_20, sparse_21,
              sparse_22, sparse_23, sparse_24, sparse_25]
    idx = jnp.stack(sparse, axis=1)
    idx_win = idx.reshape(_B // _BGRP, _BGRP, _N_SPARSE).transpose(0, 2, 1)
    x4 = _sc_gather4(tables, idx_win)
    return _tc_mlp(x4, dense_0, W1, b1, W2, b2, Wout, bout)


# speedup vs baseline: 1.6714x; 1.2418x over previous
import functools

import jax
import jax.numpy as jnp
from jax.experimental import pallas as pl
from jax.experimental.pallas import tpu as pltpu
from jax.experimental.pallas import tpu_sc as plsc

_N_SPARSE = 26
_VOCAB = 100000
_EMB = 16
_B = 16384
_DENSE = 13
_H1, _H2, _OUT = 256, 128, 64
_FPAD = 32
_NQ = 4
_BGRP = 16
_NWIN = _NQ * (_B // _BGRP)
_BM = 1024


_VCHUNK = 8192
_NVC = -(-_VOCAB // _VCHUNK)
_VPAD = _NVC * _VCHUNK


def _repack_body(in_ref, out_ref):
    x = in_ref[0]
    y = jnp.transpose(x)
    y3 = y.reshape(_VCHUNK // 8, 8, _EMB)
    for j in range(8):
        out_ref[:, pl.ds(j * _EMB, _EMB)] = y3[:, j, :]


def _tc_repack_tables(tables):
    tab_t = jnp.transpose(tables, (0, 2, 1))
    nrows = _VCHUNK * _EMB // 128
    lin = pl.pallas_call(
        _repack_body,
        grid=(_N_SPARSE, _NVC),
        in_specs=[pl.BlockSpec((1, _EMB, _VCHUNK), lambda f, c: (f, 0, c))],
        out_specs=pl.BlockSpec((nrows, 128),
                               lambda f, c: (f * _NVC + c, 0)),
        out_shape=jax.ShapeDtypeStruct((_N_SPARSE * _NVC * nrows, 128),
                                       jnp.float32),
    )(tab_t)
    return lin.reshape(_N_SPARSE, _VPAD, _EMB)


def _sc_gather4(tab_flat, idx_win):
    mesh = plsc.VectorSubcoreMesh(core_axis_name="core", subcore_axis_name="subcore")

    @functools.partial(
        pl.kernel,
        out_type=jax.ShapeDtypeStruct((_NQ, _B, 8 * _EMB), jnp.float32),
        mesh=mesh,
        scratch_types=[pltpu.VMEM((_N_SPARSE, _BGRP, _EMB), jnp.float32),
                       pltpu.SemaphoreType.DMA],
        compiler_params=pltpu.CompilerParams(use_tc_tiling_on_sc=False),
    )
    def k(tab_hbm, idx_hbm, o_hbm, gbuf, sem):
        zeros = jnp.zeros((_EMB,), jnp.float32)

        def body(i_vmem, o_vmem):
            handles = [
                pltpu.async_copy(
                    tab_hbm.at[f].at[i_vmem.at[f, 0]], gbuf.at[f], sem)
                for f in range(_N_SPARSE)
            ]
            for h in handles:
                h.wait()
            for j in range(_BGRP):
                for f in range(_N_SPARSE):
                    o_vmem[f // 8, j, pl.ds((f % 8) * _EMB, _EMB)] = gbuf[f, j, :]
                for s in range(_N_SPARSE % 8, 8):
                    o_vmem[_NQ - 1, j, pl.ds(s * _EMB, _EMB)] = zeros

        pltpu.emit_pipeline(
            body,
            grid=(_B // _BGRP,),
            in_specs=[pl.BlockSpec((_N_SPARSE, 1, _BGRP),
                                   index_map=lambda w: (0, w, 0))],
            out_specs=[pl.BlockSpec((_NQ, _BGRP, 8 * _EMB),
                                    index_map=lambda w: (0, w, 0))],
            core_axis_name=("core", "subcore"),
            dimension_semantics=(pltpu.PARALLEL,),
        )(idx_hbm, o_hbm)

    return k(tab_flat, idx_win)


def _mlp_body(x4_ref, dense_ref, w1e_ref, w1b_ref, b1_ref, w2_ref, b2_ref,
              wo_ref, bo_ref, o_ref):
    dn = (((1,), (0,)), ((), ()))
    f32 = jnp.float32
    h = jax.lax.dot_general(x4_ref[0], w1e_ref[pl.ds(0, 128), :], dn,
                            preferred_element_type=f32)
    for q in range(1, _NQ):
        h = h + jax.lax.dot_general(x4_ref[q], w1e_ref[pl.ds(128 * q, 128), :],
                                    dn, preferred_element_type=f32)
    h = h + jax.lax.dot_general(dense_ref[...], w1b_ref[...], dn,
                                preferred_element_type=f32)
    h = jnp.maximum(h + b1_ref[...], 0.0)
    h = jax.lax.dot_general(h, w2_ref[...], dn, preferred_element_type=f32)
    h = jnp.maximum(h + b2_ref[...], 0.0)
    out = jax.lax.dot_general(h, wo_ref[...], dn, preferred_element_type=f32)
    out = out + bo_ref[...]
    ssq = jnp.sum(out * out, axis=1, keepdims=True)
    denom = jnp.maximum(jnp.sqrt(ssq), 1e-12)
    o_ref[...] = out / denom


def _tc_mlp(x4, dense_0, W1, b1, W2, b2, Wout, bout):
    w1a = W1[:_N_SPARSE * _EMB].reshape(_N_SPARSE, _EMB, _H1)
    w1e = jnp.pad(w1a, ((0, _FPAD - _N_SPARSE), (0, 0), (0, 0))).reshape(
        _FPAD * _EMB, _H1)
    w1b = W1[_N_SPARSE * _EMB:]
    full = lambda shape: pl.BlockSpec(shape, lambda i: tuple(0 for _ in shape))
    return pl.pallas_call(
        _mlp_body,
        grid=(_B // _BM,),
        in_specs=[
            pl.BlockSpec((_NQ, _BM, 128), lambda i: (0, i, 0)),
            pl.BlockSpec((_BM, _DENSE), lambda i: (i, 0)),
            full((_FPAD * _EMB, _H1)),
            full((_DENSE, _H1)),
            full((1, _H1)),
            full((_H1, _H2)),
            full((1, _H2)),
            full((_H2, _OUT)),
            full((1, _OUT)),
        ],
        out_specs=pl.BlockSpec((_BM, _OUT), lambda i: (i, 0)),
        out_shape=jax.ShapeDtypeStruct((_B, _OUT), jnp.float32),
    )(x4, dense_0, w1e, w1b, b1[None, :], W2, b2[None, :], Wout, bout[None, :])


def kernel(sparse_0, sparse_1, sparse_2, sparse_3, sparse_4, sparse_5,
           sparse_6, sparse_7, sparse_8, sparse_9, sparse_10, sparse_11,
           sparse_12, sparse_13, sparse_14, sparse_15, sparse_16, sparse_17,
           sparse_18, sparse_19, sparse_20, sparse_21, sparse_22, sparse_23,
           sparse_24, sparse_25, dense_0, tables, W1, b1, W2, b2, Wout, bout):
    sparse = [sparse_0, sparse_1, sparse_2, sparse_3, sparse_4, sparse_5,
              sparse_6, sparse_7, sparse_8, sparse_9, sparse_10, sparse_11,
              sparse_12, sparse_13, sparse_14, sparse_15, sparse_16,
              sparse_17, sparse_18, sparse_19, sparse_20, sparse_21,
              sparse_22, sparse_23, sparse_24, sparse_25]
    idx_win = jnp.concatenate(sparse).reshape(_N_SPARSE, _B // _BGRP, _BGRP)
    tab_lin = _tc_repack_tables(tables)
    x4 = _sc_gather4(tab_lin, idx_win)
    return _tc_mlp(x4, dense_0, W1, b1, W2, b2, Wout, bout)


# speedup vs baseline: 1.6860x; 1.0087x over previous
import functools

import jax
import jax.numpy as jnp
from jax.experimental import pallas as pl
from jax.experimental.pallas import tpu as pltpu
from jax.experimental.pallas import tpu_sc as plsc

_N_SPARSE = 26
_VOCAB = 100000
_EMB = 16
_B = 16384
_DENSE = 13
_H1, _H2, _OUT = 256, 128, 64
_FPAD = 32
_NQ = 4
_BGRP = 16
_NWIN = _NQ * (_B // _BGRP)
_BM = 1024


_VCHUNK = 8192
_NVC = -(-_VOCAB // _VCHUNK)
_FGRP = 2
_VPAD = _NVC * _VCHUNK


def _repack_body(in_ref, out_ref):
    sub = _VCHUNK // 8
    for g in range(_FGRP):
        x = in_ref[g]
        for c in range(8):
            out_ref[g, :, pl.ds(c * _EMB, _EMB)] = jnp.transpose(
                x[:, c * sub:(c + 1) * sub])


def _tc_repack_tables(tables):
    tab_t = jnp.transpose(tables, (0, 2, 1))
    nrows = _VCHUNK * _EMB // 128
    lin = pl.pallas_call(
        _repack_body,
        grid=(_N_SPARSE // _FGRP, _NVC),
        in_specs=[pl.BlockSpec((_FGRP, _EMB, _VCHUNK),
                               lambda f, c: (f, 0, c))],
        out_specs=pl.BlockSpec((_FGRP, nrows, 128),
                               lambda f, c: (f, c, 0)),
        out_shape=jax.ShapeDtypeStruct(
            (_N_SPARSE, _NVC * nrows, 128), jnp.float32),
    )(tab_t)
    return lin.reshape(_N_SPARSE, _VPAD, _EMB)


def _sc_gather4(tab_flat, idx_win):
    mesh = plsc.VectorSubcoreMesh(core_axis_name="core", subcore_axis_name="subcore")

    @functools.partial(
        pl.kernel,
        out_type=jax.ShapeDtypeStruct((_NQ, _B, 8 * _EMB), jnp.float32),
        mesh=mesh,
        scratch_types=[pltpu.VMEM((_N_SPARSE, _BGRP, _EMB), jnp.float32),
                       pltpu.SemaphoreType.DMA],
        compiler_params=pltpu.CompilerParams(use_tc_tiling_on_sc=False),
    )
    def k(tab_hbm, idx_hbm, o_hbm, gbuf, sem):
        zeros = jnp.zeros((_EMB,), jnp.float32)

        def body(i_vmem, o_vmem):
            handles = [
                pltpu.async_copy(
                    tab_hbm.at[f].at[i_vmem.at[f, 0]], gbuf.at[f], sem)
                for f in range(_N_SPARSE)
            ]
            for h in handles:
                h.wait()
            for j in range(_BGRP):
                for f in range(_N_SPARSE):
                    o_vmem[f // 8, j, pl.ds((f % 8) * _EMB, _EMB)] = gbuf[f, j, :]
                for s in range(_N_SPARSE % 8, 8):
                    o_vmem[_NQ - 1, j, pl.ds(s * _EMB, _EMB)] = zeros

        pltpu.emit_pipeline(
            body,
            grid=(_B // _BGRP,),
            in_specs=[pl.BlockSpec((_N_SPARSE, 1, _BGRP),
                                   index_map=lambda w: (0, w, 0))],
            out_specs=[pl.BlockSpec((_NQ, _BGRP, 8 * _EMB),
                                    index_map=lambda w: (0, w, 0))],
            core_axis_name=("core", "subcore"),
            dimension_semantics=(pltpu.PARALLEL,),
        )(idx_hbm, o_hbm)

    return k(tab_flat, idx_win)


def _mlp_body(x4_ref, dense_ref, w1e_ref, w1b_ref, b1_ref, w2_ref, b2_ref,
              wo_ref, bo_ref, o_ref):
    dn = (((1,), (0,)), ((), ()))
    f32 = jnp.float32
    h = jax.lax.dot_general(x4_ref[0], w1e_ref[pl.ds(0, 128), :], dn,
                            preferred_element_type=f32)
    for q in range(1, _NQ):
        h = h + jax.lax.dot_general(x4_ref[q], w1e_ref[pl.ds(128 * q, 128), :],
                                    dn, preferred_element_type=f32)
    h = h + jax.lax.dot_general(dense_ref[...], w1b_ref[...], dn,
                                preferred_element_type=f32)
    h = jnp.maximum(h + b1_ref[...], 0.0)
    h = jax.lax.dot_general(h, w2_ref[...], dn, preferred_element_type=f32)
    h = jnp.maximum(h + b2_ref[...], 0.0)
    out = jax.lax.dot_general(h, wo_ref[...], dn, preferred_element_type=f32)
    out = out + bo_ref[...]
    ssq = jnp.sum(out * out, axis=1, keepdims=True)
    denom = jnp.maximum(jnp.sqrt(ssq), 1e-12)
    o_ref[...] = out / denom


def _tc_mlp(x4, dense_0, W1, b1, W2, b2, Wout, bout):
    w1a = W1[:_N_SPARSE * _EMB].reshape(_N_SPARSE, _EMB, _H1)
    w1e = jnp.pad(w1a, ((0, _FPAD - _N_SPARSE), (0, 0), (0, 0))).reshape(
        _FPAD * _EMB, _H1)
    w1b = W1[_N_SPARSE * _EMB:]
    full = lambda shape: pl.BlockSpec(shape, lambda i: tuple(0 for _ in shape))
    return pl.pallas_call(
        _mlp_body,
        grid=(_B // _BM,),
        in_specs=[
            pl.BlockSpec((_NQ, _BM, 128), lambda i: (0, i, 0)),
            pl.BlockSpec((_BM, _DENSE), lambda i: (i, 0)),
            full((_FPAD * _EMB, _H1)),
            full((_DENSE, _H1)),
            full((1, _H1)),
            full((_H1, _H2)),
            full((1, _H2)),
            full((_H2, _OUT)),
            full((1, _OUT)),
        ],
        out_specs=pl.BlockSpec((_BM, _OUT), lambda i: (i, 0)),
        out_shape=jax.ShapeDtypeStruct((_B, _OUT), jnp.float32),
    )(x4, dense_0, w1e, w1b, b1[None, :], W2, b2[None, :], Wout, bout[None, :])


def kernel(sparse_0, sparse_1, sparse_2, sparse_3, sparse_4, sparse_5,
           sparse_6, sparse_7, sparse_8, sparse_9, sparse_10, sparse_11,
           sparse_12, sparse_13, sparse_14, sparse_15, sparse_16, sparse_17,
           sparse_18, sparse_19, sparse_20, sparse_21, sparse_22, sparse_23,
           sparse_24, sparse_25, dense_0, tables, W1, b1, W2, b2, Wout, bout):
    sparse = [sparse_0, sparse_1, sparse_2, sparse_3, sparse_4, sparse_5,
              sparse_6, sparse_7, sparse_8, sparse_9, sparse_10, sparse_11,
              sparse_12, sparse_13, sparse_14, sparse_15, sparse_16,
              sparse_17, sparse_18, sparse_19, sparse_20, sparse_21,
              sparse_22, sparse_23, sparse_24, sparse_25]
    cat = jnp.concatenate(sparse)
    sub_bits = (_VCHUNK // 8).bit_length() - 1
    t = cat & (_VCHUNK - 1)
    cat = (cat & ~(_VCHUNK - 1)) + ((t & (_VCHUNK // 8 - 1)) << 3) + (
        t >> sub_bits)
    idx_win = cat.reshape(_N_SPARSE, _B // _BGRP, _BGRP)
    tab_lin = _tc_repack_tables(tables)
    x4 = _sc_gather4(tab_lin, idx_win)
    return _tc_mlp(x4, dense_0, W1, b1, W2, b2, Wout, bout)


# speedup vs baseline: 1.8672x; 1.1075x over previous
import functools

import jax
import jax.numpy as jnp
from jax.experimental import pallas as pl
from jax.experimental.pallas import tpu as pltpu
from jax.experimental.pallas import tpu_sc as plsc

_N_SPARSE = 26
_VOCAB = 100000
_EMB = 16
_B = 16384
_DENSE = 13
_H1, _H2, _OUT = 256, 128, 64
_FPAD = 32
_NQ = 4
_BGRP = 16
_NWIN = _NQ * (_B // _BGRP)
_BM = 1024


_VCHUNK = 8192
_NVC = -(-_VOCAB // _VCHUNK)
_FGRP = 2
_VPAD = _NVC * _VCHUNK


def _repack_body(in_ref, out_ref):
    sub = _VCHUNK // 8
    for g in range(_FGRP):
        x = in_ref[g]
        for c in range(8):
            out_ref[g, :, pl.ds(c * _EMB, _EMB)] = jnp.transpose(
                x[:, c * sub:(c + 1) * sub])


def _tc_repack_tables(tables, f0, nf):
    tab_t = jnp.transpose(tables, (0, 2, 1))
    nrows = _VCHUNK * _EMB // 128
    g0 = f0 // _FGRP
    lin = pl.pallas_call(
        _repack_body,
        grid=(nf // _FGRP, _NVC),
        in_specs=[pl.BlockSpec((_FGRP, _EMB, _VCHUNK),
                               lambda f, c: (f + g0, 0, c))],
        out_specs=pl.BlockSpec((_FGRP, nrows, 128),
                               lambda f, c: (f, c, 0)),
        out_shape=jax.ShapeDtypeStruct(
            (nf, _NVC * nrows, 128), jnp.float32),
    )(tab_t)
    return lin.reshape(nf, _VPAD, _EMB)


def _sc_gather_slabs(tab_lin, idx_win, nf):
    mesh = plsc.VectorSubcoreMesh(core_axis_name="core", subcore_axis_name="subcore")

    @functools.partial(
        pl.kernel,
        out_type=jax.ShapeDtypeStruct((2, _B, 8 * _EMB), jnp.float32),
        mesh=mesh,
        scratch_types=[pltpu.VMEM((nf, _BGRP, _EMB), jnp.float32),
                       pltpu.SemaphoreType.DMA],
        compiler_params=pltpu.CompilerParams(use_tc_tiling_on_sc=False),
    )
    def k(tab_hbm, idx_hbm, o_hbm, gbuf, sem):
        zeros = jnp.zeros((_EMB,), jnp.float32)

        def body(i_vmem, o_vmem):
            handles = [
                pltpu.async_copy(
                    tab_hbm.at[f].at[i_vmem.at[f, 0]], gbuf.at[f], sem)
                for f in range(nf)
            ]
            for h in handles:
                h.wait()
            for j in range(_BGRP):
                for f in range(nf):
                    o_vmem[f // 8, j, pl.ds((f % 8) * _EMB, _EMB)] = gbuf[f, j, :]
                for s in range(nf - 8, 8):
                    o_vmem[1, j, pl.ds(s * _EMB, _EMB)] = zeros

        pltpu.emit_pipeline(
            body,
            in_specs=[pl.BlockSpec((nf, 1, _BGRP),
                                   index_map=lambda w: (0, w, 0))],
            out_specs=[pl.BlockSpec((2, _BGRP, 8 * _EMB),
                                    index_map=lambda w: (0, w, 0))],
            grid=(_B // _BGRP,),
            core_axis_name=("core", "subcore"),
            dimension_semantics=(pltpu.PARALLEL,),
        )(idx_hbm, o_hbm)

    return k(tab_lin, idx_win)


def _mlp_body(x4a_ref, x4b_ref, dense_ref, w1e_ref, w1b_ref, b1_ref,
              w2_ref, b2_ref, wo_ref, bo_ref, o_ref):
    dn = (((1,), (0,)), ((), ()))
    f32 = jnp.float32
    slabs = [x4a_ref[0], x4a_ref[1], x4b_ref[0], x4b_ref[1]]
    h = jax.lax.dot_general(slabs[0], w1e_ref[pl.ds(0, 128), :], dn,
                            preferred_element_type=f32)
    for q in range(1, _NQ):
        h = h + jax.lax.dot_general(slabs[q], w1e_ref[pl.ds(128 * q, 128), :],
                                    dn, preferred_element_type=f32)
    h = h + jax.lax.dot_general(dense_ref[...], w1b_ref[...], dn,
                                preferred_element_type=f32)
    h = jnp.maximum(h + b1_ref[...], 0.0)
    h = jax.lax.dot_general(h, w2_ref[...], dn, preferred_element_type=f32)
    h = jnp.maximum(h + b2_ref[...], 0.0)
    out = jax.lax.dot_general(h, wo_ref[...], dn, preferred_element_type=f32)
    out = out + bo_ref[...]
    ssq = jnp.sum(out * out, axis=1, keepdims=True)
    denom = jnp.maximum(jnp.sqrt(ssq), 1e-12)
    o_ref[...] = out / denom


def _tc_mlp(x4a, x4b, dense_0, W1, b1, W2, b2, Wout, bout):
    w1a = W1[:_N_SPARSE * _EMB].reshape(_N_SPARSE, _EMB, _H1)
    w1e = jnp.pad(w1a, ((0, _FPAD - _N_SPARSE), (0, 0), (0, 0))).reshape(
        _FPAD * _EMB, _H1)
    w1b = W1[_N_SPARSE * _EMB:]
    full = lambda shape: pl.BlockSpec(shape, lambda i: tuple(0 for _ in shape))
    return pl.pallas_call(
        _mlp_body,
        grid=(_B // _BM,),
        in_specs=[
            pl.BlockSpec((2, _BM, 128), lambda i: (0, i, 0)),
            pl.BlockSpec((2, _BM, 128), lambda i: (0, i, 0)),
            pl.BlockSpec((_BM, _DENSE), lambda i: (i, 0)),
            full((_FPAD * _EMB, _H1)),
            full((_DENSE, _H1)),
            full((1, _H1)),
            full((_H1, _H2)),
            full((1, _H2)),
            full((_H2, _OUT)),
            full((1, _OUT)),
        ],
        out_specs=pl.BlockSpec((_BM, _OUT), lambda i: (i, 0)),
        out_shape=jax.ShapeDtypeStruct((_B, _OUT), jnp.float32),
    )(x4a, x4b, dense_0, w1e, w1b, b1[None, :], W2, b2[None, :], Wout,
      bout[None, :])


def kernel(sparse_0, sparse_1, sparse_2, sparse_3, sparse_4, sparse_5,
           sparse_6, sparse_7, sparse_8, sparse_9, sparse_10, sparse_11,
           sparse_12, sparse_13, sparse_14, sparse_15, sparse_16, sparse_17,
           sparse_18, sparse_19, sparse_20, sparse_21, sparse_22, sparse_23,
           sparse_24, sparse_25, dense_0, tables, W1, b1, W2, b2, Wout, bout):
    sparse = [sparse_0, sparse_1, sparse_2, sparse_3, sparse_4, sparse_5,
              sparse_6, sparse_7, sparse_8, sparse_9, sparse_10, sparse_11,
              sparse_12, sparse_13, sparse_14, sparse_15, sparse_16,
              sparse_17, sparse_18, sparse_19, sparse_20, sparse_21,
              sparse_22, sparse_23, sparse_24, sparse_25]
    sub_bits = (_VCHUNK // 8).bit_length() - 1

    def permute(fields):
        cat = jnp.concatenate(fields)
        t = cat & (_VCHUNK - 1)
        cat = (cat & ~(_VCHUNK - 1)) + ((t & (_VCHUNK // 8 - 1)) << 3) + (
            t >> sub_bits)
        return cat.reshape(len(fields), _B // _BGRP, _BGRP)

    tab_a = _tc_repack_tables(tables, 0, 16)
    tab_b = _tc_repack_tables(tables, 16, 10)
    x4a = _sc_gather_slabs(tab_a, permute(sparse[:16]), 16)
    x4b = _sc_gather_slabs(tab_b, permute(sparse[16:]), 10)
    return _tc_mlp(x4a, x4b, dense_0, W1, b1, W2, b2, Wout, bout)
